# baseline (speedup 1.0000x reference)
import functools

import jax
import jax.numpy as jnp
from jax import lax
from jax.experimental import pallas as pl
from jax.experimental.pallas import tpu as pltpu
from jax.experimental.pallas import tpu_sc as plsc

EMB = 32
HID = 128

_FIRE = 128
_FPG = 8
_GROUP = _FIRE * _FPG

_TC = 800


def _mesh_info():
    info = plsc.get_sparse_core_info()
    return info.num_cores, info.num_subcores


def _sc_transpose(wt):
    v = wt.shape[1]
    nc, ns = _mesh_info()
    nw = nc * ns
    nchunks = v // _TC
    tmax = (nchunks + nw - 1) // nw
    mesh = plsc.VectorSubcoreMesh(core_axis_name="c", subcore_axis_name="s")
    lanes = 16

    @functools.partial(
        pl.kernel,
        mesh=mesh,
        out_type=jax.ShapeDtypeStruct((v, EMB), jnp.float32),
        scratch_types=[
            pltpu.VMEM((2, EMB, _TC), jnp.float32),
            pltpu.VMEM((2, _TC, EMB), jnp.float32),
            pltpu.SemaphoreType.DMA,
            pltpu.SemaphoreType.DMA,
            pltpu.SemaphoreType.DMA,
            pltpu.SemaphoreType.DMA,
        ],
        compiler_params=pltpu.CompilerParams(
            use_tc_tiling_on_sc=False, needs_layout_passes=False
        ),
    )
    def transpose(wt_hbm, out_hbm, ibuf, obuf, sem_i0, sem_i1, sem_o0, sem_o1):
        wid = lax.axis_index("s") * nc + lax.axis_index("c")

        def stage_in(slot, t, sem_i):
            c = wid + t * nw

            @pl.when(c < nchunks)
            def _():
                pltpu.async_copy(
                    wt_hbm.at[:, pl.ds(c * _TC, _TC)],
                    ibuf.at[slot],
                    sem_i,
                )

        def do_chunk(slot, t, sem_i, sem_o):
            c = wid + t * nw

            @pl.when(c < nchunks)
            def _():
                pltpu.make_async_copy(
                    wt_hbm.at[:, pl.ds(0, _TC)], ibuf.at[slot], sem_i
                ).wait()

                @pl.when(t >= 2)
                def _():
                    pltpu.make_async_copy(
                        obuf.at[slot], out_hbm.at[pl.ds(0, _TC)], sem_o
                    ).wait()

                def vgroup(i, carry):
                    v16 = i * lanes
                    row_idx = v16 + lax.iota(jnp.int32, lanes)
                    for e in range(EMB):
                        vals = ibuf[slot, e, pl.ds(v16, lanes)]
                        plsc.store_scatter(
                            obuf.at[slot],
                            [row_idx, jnp.full((lanes,), e, jnp.int32)],
                            vals,
                        )
                    return carry

                lax.fori_loop(0, _TC // lanes, vgroup, 0)

                pltpu.async_copy(
                    obuf.at[slot], out_hbm.at[pl.ds(c * _TC, _TC)], sem_o
                )

        def body(i, carry):
            stage_in(1, 2 * i + 1, sem_i1)
            do_chunk(0, 2 * i, sem_i0, sem_o0)
            stage_in(0, 2 * i + 2, sem_i0)
            do_chunk(1, 2 * i + 1, sem_i1, sem_o1)
            return carry

        stage_in(0, 0, sem_i0)
        lax.fori_loop(0, (tmax + 1) // 2, body, 0)

        pltpu.make_async_copy(obuf.at[0], out_hbm.at[pl.ds(0, _TC)], sem_o0).wait()
        pltpu.make_async_copy(obuf.at[1], out_hbm.at[pl.ds(0, _TC)], sem_o1).wait()

    return transpose(wt)


def _sc_gather(x2d, w):
    n = x2d.shape[0] * x2d.shape[1]
    nc, ns = _mesh_info()
    nw = nc * ns
    per_w = n // nw
    groups = per_w // _GROUP

    mesh = plsc.VectorSubcoreMesh(core_axis_name="c", subcore_axis_name="s")

    @functools.partial(
        pl.kernel,
        mesh=mesh,
        out_type=jax.ShapeDtypeStruct((n, HID), jnp.float32),
        scratch_types=[
            pltpu.VMEM((2, _FPG, _FIRE), jnp.int32),
            pltpu.VMEM((2, _GROUP, EMB), jnp.float32),
            pltpu.SemaphoreType.DMA,
            pltpu.SemaphoreType.DMA,
            pltpu.SemaphoreType.DMA,
        ],
        compiler_params=pltpu.CompilerParams(use_tc_tiling_on_sc=False),
    )
    def gather(x_hbm, w_hbm, out_hbm, idx_v, rows_v, sem_g, sem_i, sem_o):
        wid = lax.axis_index("s") * nc + lax.axis_index("c")
        row_base = wid * (per_w // _FIRE)
        out_base = wid * per_w

        def fire_group(slot):
            for j in range(_FPG):
                pltpu.async_copy(
                    w_hbm.at[idx_v.at[slot, j]],
                    rows_v.at[slot, pl.ds(j * _FIRE, _FIRE)],
                    sem_g,
                )

        pltpu.sync_copy(x_hbm.at[pl.ds(row_base, _FPG)], idx_v.at[0])
        fire_group(0)

        def body(g, carry):
            slot = lax.rem(g, 2)
            nslot = 1 - slot

            @pl.when(g + 1 < groups)
            def _():
                pltpu.async_copy(
                    x_hbm.at[pl.ds(row_base + (g + 1) * _FPG, _FPG)],
                    idx_v.at[nslot],
                    sem_i,
                )

            pltpu.make_async_copy(
                out_hbm.at[pl.ds(out_base, _GROUP), pl.ds(0, EMB)],
                rows_v.at[slot],
                sem_g,
            ).wait()

            @pl.when(g >= 1)
            def _():
                pltpu.make_async_copy(
                    rows_v.at[nslot],
                    out_hbm.at[pl.ds(out_base, _GROUP), pl.ds(0, EMB)],
                    sem_o,
                ).wait()

            @pl.when(g + 1 < groups)
            def _():
                pltpu.make_async_copy(
                    x_hbm.at[pl.ds(row_base, _FPG)],
                    idx_v.at[nslot],
                    sem_i,
                ).wait()
                fire_group(nslot)

            pltpu.async_copy(
                rows_v.at[slot],
                out_hbm.at[pl.ds(out_base + g * _GROUP, _GROUP), pl.ds(0, EMB)],
                sem_o,
            )
            return carry

        lax.fori_loop(0, groups, body, 0)

        pltpu.make_async_copy(
            rows_v.at[(groups - 1) % 2],
            out_hbm.at[pl.ds(out_base, _GROUP), pl.ds(0, EMB)],
            sem_o,
        ).wait()

    return gather(x2d, w)


def _tc_project(g, we, b, l):
    bb = 16
    rows = bb * l

    def mm(g_ref, we_ref, o_ref):
        acc = lax.dot_general(
            g_ref[:, :EMB],
            we_ref[...],
            (((1,), (1,)), ((), ())),
            preferred_element_type=jnp.float32,
        )
        o_ref[...] = acc.reshape(bb, l, HID)

    return pl.pallas_call(
        mm,
        grid=(b // bb,),
        in_specs=[
            pl.BlockSpec((rows, HID), lambda i: (i, 0)),
            pl.BlockSpec((HID, EMB), lambda i: (0, 0)),
        ],
        out_specs=pl.BlockSpec((bb, l, HID), lambda i: (i, 0, 0)),
        out_shape=jax.ShapeDtypeStruct((b, l, HID), jnp.float32),
    )(g, we)


def kernel(x, W, We):
    b, l = x.shape
    n = b * l
    x2d = x.reshape(n // _FIRE, _FIRE).astype(jnp.int32)
    w_lin = _sc_transpose(jnp.transpose(W))
    g = _sc_gather(x2d, w_lin)
    return _tc_project(g, We, b, l)


# speedup vs baseline: 1.0459x; 1.0459x over previous
import functools

import jax
import jax.numpy as jnp
from jax import lax
from jax.experimental import pallas as pl
from jax.experimental.pallas import tpu as pltpu
from jax.experimental.pallas import tpu_sc as plsc

EMB = 32
HID = 128

_FIRE = 128
_FPG = 8
_GROUP = _FIRE * _FPG

_TC = 800


def _mesh_info():
    info = plsc.get_sparse_core_info()
    return info.num_cores, info.num_subcores


def _sc_transpose(wt):
    v = wt.shape[1]
    nc, ns = _mesh_info()
    nw = nc * ns
    nchunks = v // _TC
    tmax = (nchunks + nw - 1) // nw
    mesh = plsc.VectorSubcoreMesh(core_axis_name="c", subcore_axis_name="s")
    lanes = 16

    @functools.partial(
        pl.kernel,
        mesh=mesh,
        out_type=jax.ShapeDtypeStruct((v, EMB), jnp.float32),
        scratch_types=[
            pltpu.VMEM((2, EMB, _TC), jnp.float32),
            pltpu.VMEM((2, _TC, EMB), jnp.float32),
            pltpu.SemaphoreType.DMA,
            pltpu.SemaphoreType.DMA,
            pltpu.SemaphoreType.DMA,
            pltpu.SemaphoreType.DMA,
        ],
        compiler_params=pltpu.CompilerParams(
            use_tc_tiling_on_sc=False, needs_layout_passes=False
        ),
    )
    def transpose(wt_hbm, out_hbm, ibuf, obuf, sem_i0, sem_i1, sem_o0, sem_o1):
        wid = lax.axis_index("s") * nc + lax.axis_index("c")

        def stage_in(slot, t, sem_i):
            c = wid + t * nw

            @pl.when(c < nchunks)
            def _():
                pltpu.async_copy(
                    wt_hbm.at[:, pl.ds(c * _TC, _TC)],
                    ibuf.at[slot],
                    sem_i,
                )

        def do_chunk(slot, t, sem_i, sem_o):
            c = wid + t * nw

            @pl.when(c < nchunks)
            def _():
                pltpu.make_async_copy(
                    wt_hbm.at[:, pl.ds(0, _TC)], ibuf.at[slot], sem_i
                ).wait()

                @pl.when(t >= 2)
                def _():
                    pltpu.make_async_copy(
                        obuf.at[slot], out_hbm.at[pl.ds(0, _TC)], sem_o
                    ).wait()

                @plsc.parallel_loop(0, _TC // lanes, 1)
                def vgroup(i):
                    v16 = i * lanes
                    row_idx = v16 + lax.iota(jnp.int32, lanes)
                    vals = [
                        ibuf[slot, e, pl.ds(v16, lanes)] for e in range(EMB)
                    ]
                    for e in range(EMB):
                        plsc.store_scatter(
                            obuf.at[slot],
                            [row_idx, jnp.full((lanes,), e, jnp.int32)],
                            vals[e],
                        )

                pltpu.async_copy(
                    obuf.at[slot], out_hbm.at[pl.ds(c * _TC, _TC)], sem_o
                )

        def body(i, carry):
            stage_in(1, 2 * i + 1, sem_i1)
            do_chunk(0, 2 * i, sem_i0, sem_o0)
            stage_in(0, 2 * i + 2, sem_i0)
            do_chunk(1, 2 * i + 1, sem_i1, sem_o1)
            return carry

        stage_in(0, 0, sem_i0)
        lax.fori_loop(0, (tmax + 1) // 2, body, 0)

        pltpu.make_async_copy(obuf.at[0], out_hbm.at[pl.ds(0, _TC)], sem_o0).wait()
        pltpu.make_async_copy(obuf.at[1], out_hbm.at[pl.ds(0, _TC)], sem_o1).wait()

    return transpose(wt)


def _sc_gather(x2d, w):
    n = x2d.shape[0] * x2d.shape[1]
    nc, ns = _mesh_info()
    nw = nc * ns
    per_w = n // nw
    groups = per_w // _GROUP

    mesh = plsc.VectorSubcoreMesh(core_axis_name="c", subcore_axis_name="s")

    @functools.partial(
        pl.kernel,
        mesh=mesh,
        out_type=jax.ShapeDtypeStruct((n, HID), jnp.float32),
        scratch_types=[
            pltpu.VMEM((2, _FPG, _FIRE), jnp.int32),
            pltpu.VMEM((2, _GROUP, EMB), jnp.float32),
            pltpu.SemaphoreType.DMA,
            pltpu.SemaphoreType.DMA,
            pltpu.SemaphoreType.DMA,
        ],
        compiler_params=pltpu.CompilerParams(use_tc_tiling_on_sc=False),
    )
    def gather(x_hbm, w_hbm, out_hbm, idx_v, rows_v, sem_g, sem_i, sem_o):
        wid = lax.axis_index("s") * nc + lax.axis_index("c")
        row_base = wid * (per_w // _FIRE)
        out_base = wid * per_w

        def fire_group(slot):
            for j in range(_FPG):
                pltpu.async_copy(
                    w_hbm.at[idx_v.at[slot, j]],
                    rows_v.at[slot, pl.ds(j * _FIRE, _FIRE)],
                    sem_g,
                )

        pltpu.sync_copy(x_hbm.at[pl.ds(row_base, _FPG)], idx_v.at[0])
        fire_group(0)

        def body(g, carry):
            slot = lax.rem(g, 2)
            nslot = 1 - slot

            @pl.when(g + 1 < groups)
            def _():
                pltpu.async_copy(
                    x_hbm.at[pl.ds(row_base + (g + 1) * _FPG, _FPG)],
                    idx_v.at[nslot],
                    sem_i,
                )

            pltpu.make_async_copy(
                out_hbm.at[pl.ds(out_base, _GROUP), pl.ds(0, EMB)],
                rows_v.at[slot],
                sem_g,
            ).wait()

            @pl.when(g >= 1)
            def _():
                pltpu.make_async_copy(
                    rows_v.at[nslot],
                    out_hbm.at[pl.ds(out_base, _GROUP), pl.ds(0, EMB)],
                    sem_o,
                ).wait()

            @pl.when(g + 1 < groups)
            def _():
                pltpu.make_async_copy(
                    x_hbm.at[pl.ds(row_base, _FPG)],
                    idx_v.at[nslot],
                    sem_i,
                ).wait()
                fire_group(nslot)

            pltpu.async_copy(
                rows_v.at[slot],
                out_hbm.at[pl.ds(out_base + g * _GROUP, _GROUP), pl.ds(0, EMB)],
                sem_o,
            )
            return carry

        lax.fori_loop(0, groups, body, 0)

        pltpu.make_async_copy(
            rows_v.at[(groups - 1) % 2],
            out_hbm.at[pl.ds(out_base, _GROUP), pl.ds(0, EMB)],
            sem_o,
        ).wait()

    return gather(x2d, w)


def _tc_project(g, we, b, l):
    bb = 16
    rows = bb * l

    def mm(g_ref, we_ref, o_ref):
        acc = lax.dot_general(
            g_ref[:, :EMB],
            we_ref[...],
            (((1,), (1,)), ((), ())),
            preferred_element_type=jnp.float32,
        )
        o_ref[...] = acc.reshape(bb, l, HID)

    return pl.pallas_call(
        mm,
        grid=(b // bb,),
        in_specs=[
            pl.BlockSpec((rows, HID), lambda i: (i, 0)),
            pl.BlockSpec((HID, EMB), lambda i: (0, 0)),
        ],
        out_specs=pl.BlockSpec((bb, l, HID), lambda i: (i, 0, 0)),
        out_shape=jax.ShapeDtypeStruct((b, l, HID), jnp.float32),
    )(g, we)


def kernel(x, W, We):
    b, l = x.shape
    n = b * l
    x2d = x.reshape(n // _FIRE, _FIRE).astype(jnp.int32)
    w_lin = _sc_transpose(jnp.transpose(W))
    g = _sc_gather(x2d, w_lin)
    return _tc_project(g, We, b, l)


# speedup vs baseline: 3.8750x; 3.7049x over previous
import functools

import jax
import jax.numpy as jnp
from jax import lax
from jax.experimental import pallas as pl
from jax.experimental.pallas import tpu as pltpu
from jax.experimental.pallas import tpu_sc as plsc

EMB = 32
HID = 128

_FIRE = 128
_FPG = 8
_GROUP = _FIRE * _FPG


def _sc_gather(x2d, w):
    n = x2d.shape[0] * x2d.shape[1]
    info = plsc.get_sparse_core_info()
    nc, ns = info.num_cores, info.num_subcores
    nw = nc * ns
    per_w = n // nw
    groups = per_w // _GROUP

    mesh = plsc.VectorSubcoreMesh(core_axis_name="c", subcore_axis_name="s")

    @functools.partial(
        pl.kernel,
        mesh=mesh,
        out_type=jax.ShapeDtypeStruct((n, HID), jnp.float32),
        scratch_types=[
            pltpu.VMEM((2, _FPG, _FIRE), jnp.int32),
            pltpu.VMEM((2, _GROUP, EMB), jnp.float32),
            pltpu.SemaphoreType.DMA,
            pltpu.SemaphoreType.DMA,
            pltpu.SemaphoreType.DMA,
        ],
        compiler_params=pltpu.CompilerParams(use_tc_tiling_on_sc=False),
    )
    def gather(x_hbm, w_hbm, out_hbm, idx_v, rows_v, sem_g, sem_i, sem_o):
        wid = lax.axis_index("s") * nc + lax.axis_index("c")
        row_base = wid * (per_w // _FIRE)
        out_base = wid * per_w

        def fire_group(slot):
            for j in range(_FPG):
                pltpu.async_copy(
                    w_hbm.at[idx_v.at[slot, j]],
                    rows_v.at[slot, pl.ds(j * _FIRE, _FIRE)],
                    sem_g,
                )

        pltpu.sync_copy(x_hbm.at[pl.ds(row_base, _FPG)], idx_v.at[0])
        fire_group(0)

        def body(g, carry):
            slot = lax.rem(g, 2)
            nslot = 1 - slot

            @pl.when(g + 1 < groups)
            def _():
                pltpu.async_copy(
                    x_hbm.at[pl.ds(row_base + (g + 1) * _FPG, _FPG)],
                    idx_v.at[nslot],
                    sem_i,
                )

            pltpu.make_async_copy(
                out_hbm.at[pl.ds(out_base, _GROUP), pl.ds(0, EMB)],
                rows_v.at[slot],
                sem_g,
            ).wait()

            @pl.when(g >= 1)
            def _():
                pltpu.make_async_copy(
                    rows_v.at[nslot],
                    out_hbm.at[pl.ds(out_base, _GROUP), pl.ds(0, EMB)],
                    sem_o,
                ).wait()

            @pl.when(g + 1 < groups)
            def _():
                pltpu.make_async_copy(
                    x_hbm.at[pl.ds(row_base, _FPG)],
                    idx_v.at[nslot],
                    sem_i,
                ).wait()
                fire_group(nslot)

            pltpu.async_copy(
                rows_v.at[slot],
                out_hbm.at[pl.ds(out_base + g * _GROUP, _GROUP), pl.ds(0, EMB)],
                sem_o,
            )
            return carry

        lax.fori_loop(0, groups, body, 0)

        pltpu.make_async_copy(
            rows_v.at[(groups - 1) % 2],
            out_hbm.at[pl.ds(out_base, _GROUP), pl.ds(0, EMB)],
            sem_o,
        ).wait()

    return gather(x2d, w)


def _tc_project(g, we, b, l):
    bb = 16
    rows = bb * l

    def mm(g_ref, we_ref, o_ref):
        acc = lax.dot_general(
            g_ref[:, :EMB],
            we_ref[...],
            (((1,), (1,)), ((), ())),
            preferred_element_type=jnp.float32,
        )
        o_ref[...] = acc.reshape(bb, l, HID)

    return pl.pallas_call(
        mm,
        grid=(b // bb,),
        in_specs=[
            pl.BlockSpec((rows, HID), lambda i: (i, 0)),
            pl.BlockSpec((HID, EMB), lambda i: (0, 0)),
        ],
        out_specs=pl.BlockSpec((bb, l, HID), lambda i: (i, 0, 0)),
        out_shape=jax.ShapeDtypeStruct((b, l, HID), jnp.float32),
    )(g, we)


def kernel(x, W, We):
    b, l = x.shape
    n = b * l
    x2d = x.reshape(n // _FIRE, _FIRE).astype(jnp.int32)
    g = _sc_gather(x2d, W)
    return _tc_project(g, We, b, l)


# speedup vs baseline: 4.1630x; 1.0743x over previous
import functools

import jax
import jax.numpy as jnp
from jax import lax
from jax.experimental import pallas as pl
from jax.experimental.pallas import tpu as pltpu
from jax.experimental.pallas import tpu_sc as plsc

EMB = 32
HID = 128
_PACK = HID // EMB

_FIRE = 128
_FPG = 8
_GROUP = _FIRE * _FPG
_CHUNK = _GROUP // _PACK


def _sc_gather(x2d, w):
    n = x2d.shape[0] * x2d.shape[1]
    info = plsc.get_sparse_core_info()
    nc, ns = info.num_cores, info.num_subcores
    nw = nc * ns
    per_w = n // nw
    groups = per_w // _GROUP

    mesh = plsc.VectorSubcoreMesh(core_axis_name="c", subcore_axis_name="s")

    @functools.partial(
        pl.kernel,
        mesh=mesh,
        out_type=jax.ShapeDtypeStruct((n // _PACK, HID), jnp.float32),
        scratch_types=[
            pltpu.VMEM((2, _FPG, _FIRE), jnp.int32),
            pltpu.VMEM((2, _GROUP, EMB), jnp.float32),
            pltpu.SemaphoreType.DMA,
            pltpu.SemaphoreType.DMA,
            pltpu.SemaphoreType.DMA,
        ],
        compiler_params=pltpu.CompilerParams(use_tc_tiling_on_sc=False),
    )
    def gather(x_hbm, w_hbm, out_hbm, idx_v, rows_v, sem_g, sem_i, sem_o):
        wid = lax.axis_index("s") * nc + lax.axis_index("c")
        row_base = wid * (per_w // _FIRE)
        out_base = wid * (per_w // _PACK)

        def fire_group(slot):
            for j in range(_FPG):
                pltpu.async_copy(
                    w_hbm.at[idx_v.at[slot, j]],
                    rows_v.at[slot, pl.ds(j * _FIRE, _FIRE)],
                    sem_g,
                )

        def store_copies(slot, g):
            return [
                (
                    rows_v.at[slot, pl.ds(c * _CHUNK, _CHUNK)],
                    out_hbm.at[
                        pl.ds(out_base + g * _CHUNK, _CHUNK),
                        pl.ds(c * EMB, EMB),
                    ],
                )
                for c in range(_PACK)
            ]

        pltpu.sync_copy(x_hbm.at[pl.ds(row_base, _FPG)], idx_v.at[0])
        fire_group(0)

        def body(g, carry):
            slot = lax.rem(g, 2)
            nslot = 1 - slot

            @pl.when(g + 1 < groups)
            def _():
                pltpu.async_copy(
                    x_hbm.at[pl.ds(row_base + (g + 1) * _FPG, _FPG)],
                    idx_v.at[nslot],
                    sem_i,
                )

            pltpu.make_async_copy(
                out_hbm.at[pl.ds(0, _GROUP), pl.ds(0, EMB)],
                rows_v.at[slot],
                sem_g,
            ).wait()

            @pl.when(g >= 1)
            def _():
                for src, dst in store_copies(nslot, 0):
                    pltpu.make_async_copy(src, dst, sem_o).wait()

            @pl.when(g + 1 < groups)
            def _():
                pltpu.make_async_copy(
                    x_hbm.at[pl.ds(row_base, _FPG)],
                    idx_v.at[nslot],
                    sem_i,
                ).wait()
                fire_group(nslot)

            for src, dst in store_copies(slot, g):
                pltpu.async_copy(src, dst, sem_o)
            return carry

        lax.fori_loop(0, groups, body, 0)

        for src, dst in store_copies((groups - 1) % 2, 0):
            pltpu.make_async_copy(src, dst, sem_o).wait()

    return gather(x2d, w)


def _tc_project(g4, m, b, l):
    bb = 4
    rows = bb * l
    nsteps = (b // bb) // _PACK

    def mm(g4_ref, m_ref, o_ref):
        for k in range(_PACK):
            acc = lax.dot_general(
                g4_ref[...],
                m_ref[k],
                (((1,), (0,)), ((), ())),
                preferred_element_type=jnp.float32,
            )
            o_ref[k] = acc.reshape(bb, l, HID)

    return pl.pallas_call(
        mm,
        grid=(nsteps,),
        in_specs=[
            pl.BlockSpec((rows, HID), lambda i: (i, 0)),
            pl.BlockSpec((_PACK, HID, HID), lambda i: (0, 0, 0)),
        ],
        out_specs=pl.BlockSpec((_PACK, bb, l, HID), lambda i: (0, i, 0, 0)),
        out_shape=jax.ShapeDtypeStruct((_PACK, b // _PACK, l, HID), jnp.float32),
    )(g4, m)


def kernel(x, W, We):
    b, l = x.shape
    n = b * l
    xf = x.reshape(-1).astype(jnp.int32)
    xp = (
        xf.reshape(_PACK, (n // _PACK) // _CHUNK, _CHUNK)
        .transpose(1, 0, 2)
        .reshape(n // _FIRE, _FIRE)
    )
    g4 = _sc_gather(xp, W)
    m = jnp.stack(
        [
            jnp.zeros((HID, HID), jnp.float32)
            .at[k * EMB:(k + 1) * EMB, :]
            .set(We.T)
            for k in range(_PACK)
        ]
    )
    out4 = _tc_project(g4, m, b, l)
    return out4.reshape(b, l, HID)


# speedup vs baseline: 4.5580x; 1.0949x over previous
import functools

import jax
import jax.numpy as jnp
from jax import lax
from jax.experimental import pallas as pl
from jax.experimental.pallas import tpu as pltpu
from jax.experimental.pallas import tpu_sc as plsc

EMB = 32
HID = 128
_PACK = HID // EMB

_FIRE = 128
_FPG = 8
_GROUP = _FIRE * _FPG
_CHUNK = _GROUP // _PACK


def _sc_gather(x2d, w):
    n = x2d.shape[0] * x2d.shape[1]
    info = plsc.get_sparse_core_info()
    nc, ns = info.num_cores, info.num_subcores
    nw = nc * ns
    per_w = n // nw
    groups = per_w // _GROUP

    mesh = plsc.VectorSubcoreMesh(core_axis_name="c", subcore_axis_name="s")

    @functools.partial(
        pl.kernel,
        mesh=mesh,
        out_type=jax.ShapeDtypeStruct((n // _PACK, HID), jnp.float32),
        scratch_types=[
            pltpu.VMEM((2, _FPG, _FIRE), jnp.int32),
            pltpu.VMEM((2, _GROUP, EMB), jnp.float32),
            pltpu.SemaphoreType.DMA,
            pltpu.SemaphoreType.DMA,
            pltpu.SemaphoreType.DMA,
        ],
        compiler_params=pltpu.CompilerParams(use_tc_tiling_on_sc=False),
    )
    def gather(x_hbm, w_hbm, out_hbm, idx_v, rows_v, sem_g, sem_i, sem_o):
        wid = lax.axis_index("s") * nc + lax.axis_index("c")
        row_base = wid * (per_w // _FIRE)
        out_base = wid * (per_w // _PACK)

        def fire_group(slot):
            for j in range(_FPG):
                pltpu.async_copy(
                    w_hbm.at[idx_v.at[slot, j]],
                    rows_v.at[slot, pl.ds(j * _FIRE, _FIRE)],
                    sem_g,
                )

        def store_copies(slot, g):
            return [
                (
                    rows_v.at[slot, pl.ds(c * _CHUNK, _CHUNK)],
                    out_hbm.at[
                        pl.ds(out_base + g * _CHUNK, _CHUNK),
                        pl.ds(c * EMB, EMB),
                    ],
                )
                for c in range(_PACK)
            ]

        pltpu.sync_copy(x_hbm.at[pl.ds(row_base, _FPG)], idx_v.at[0])
        fire_group(0)

        def body(g, carry):
            slot = lax.rem(g, 2)
            nslot = 1 - slot

            @pl.when(g + 1 < groups)
            def _():
                pltpu.async_copy(
                    x_hbm.at[pl.ds(row_base + (g + 1) * _FPG, _FPG)],
                    idx_v.at[nslot],
                    sem_i,
                )

            pltpu.make_async_copy(
                out_hbm.at[pl.ds(0, _GROUP), pl.ds(0, EMB)],
                rows_v.at[slot],
                sem_g,
            ).wait()

            @pl.when(g >= 1)
            def _():
                for src, dst in store_copies(nslot, 0):
                    pltpu.make_async_copy(src, dst, sem_o).wait()

            @pl.when(g + 1 < groups)
            def _():
                pltpu.make_async_copy(
                    x_hbm.at[pl.ds(row_base, _FPG)],
                    idx_v.at[nslot],
                    sem_i,
                ).wait()
                fire_group(nslot)

            for src, dst in store_copies(slot, g):
                pltpu.async_copy(src, dst, sem_o)
            return carry

        lax.fori_loop(0, groups, body, 0)

        for src, dst in store_copies((groups - 1) % 2, 0):
            pltpu.make_async_copy(src, dst, sem_o).wait()

    return gather(x2d, w)


def _tc_project(g4, m, b, l):
    bb = 8
    rows = bb * l
    nsteps = (b // bb) // _PACK

    def mm(g4_ref, m_ref, o_ref):
        for k in range(_PACK):
            acc = lax.dot_general(
                g4_ref[...],
                m_ref[k],
                (((1,), (0,)), ((), ())),
                preferred_element_type=jnp.float32,
            )
            o_ref[k] = acc.reshape(bb, l, HID)

    return pl.pallas_call(
        mm,
        grid=(nsteps,),
        in_specs=[
            pl.BlockSpec((rows, HID), lambda i: (i, 0)),
            pl.BlockSpec((_PACK, HID, HID), lambda i: (0, 0, 0)),
        ],
        out_specs=pl.BlockSpec((_PACK, bb, l, HID), lambda i: (0, i, 0, 0)),
        out_shape=jax.ShapeDtypeStruct((_PACK, b // _PACK, l, HID), jnp.float32),
    )(g4, m)


def kernel(x, W, We):
    b, l = x.shape
    n = b * l
    xf = x.reshape(-1).astype(jnp.int32)
    xp = (
        xf.reshape(_PACK, (n // _PACK) // _CHUNK, _CHUNK)
        .transpose(1, 0, 2)
        .reshape(n // _FIRE, _FIRE)
    )
    g4 = _sc_gather(xp, W)
    m = jnp.stack(
        [
            jnp.zeros((HID, HID), jnp.float32)
            .at[k * EMB:(k + 1) * EMB, :]
            .set(We.T)
            for k in range(_PACK)
        ]
    )
    out4 = _tc_project(g4, m, b, l)
    return out4.reshape(b, l, HID)


# speedup vs baseline: 4.7516x; 1.0425x over previous
import functools

import jax
import jax.numpy as jnp
from jax import lax
from jax.experimental import pallas as pl
from jax.experimental.pallas import tpu as pltpu
from jax.experimental.pallas import tpu_sc as plsc

EMB = 32
HID = 128
_PACK = HID // EMB

_FIRE = 128
_FPG = 8
_GROUP = _FIRE * _FPG
_CHUNK = _GROUP // _PACK


def _sc_gather(x2d, w):
    n = x2d.shape[0] * x2d.shape[1]
    info = plsc.get_sparse_core_info()
    nc, ns = info.num_cores, info.num_subcores
    nw = nc * ns
    per_w = n // nw
    groups = per_w // _GROUP

    mesh = plsc.VectorSubcoreMesh(core_axis_name="c", subcore_axis_name="s")

    @functools.partial(
        pl.kernel,
        mesh=mesh,
        out_type=jax.ShapeDtypeStruct((n // _PACK, HID), jnp.float32),
        scratch_types=[
            pltpu.VMEM((2, _FPG, _FIRE), jnp.int32),
            pltpu.VMEM((2, _GROUP, EMB), jnp.float32),
            pltpu.SemaphoreType.DMA,
            pltpu.SemaphoreType.DMA,
            pltpu.SemaphoreType.DMA,
        ],
        compiler_params=pltpu.CompilerParams(use_tc_tiling_on_sc=False),
    )
    def gather(x_hbm, w_hbm, out_hbm, idx_v, rows_v, sem_g, sem_i, sem_o):
        wid = lax.axis_index("s") * nc + lax.axis_index("c")
        row_base = wid * (per_w // _FIRE)
        out_base = wid * (per_w // _PACK)

        def fire_group(slot):
            for j in range(_FPG):
                pltpu.async_copy(
                    w_hbm.at[idx_v.at[slot, j]],
                    rows_v.at[slot, pl.ds(j * _FIRE, _FIRE)],
                    sem_g,
                )

        def store_copies(slot, g):
            return [
                (
                    rows_v.at[slot, pl.ds(c * _CHUNK, _CHUNK)],
                    out_hbm.at[
                        pl.ds(out_base + g * _CHUNK, _CHUNK),
                        pl.ds(c * EMB, EMB),
                    ],
                )
                for c in range(_PACK)
            ]

        pltpu.sync_copy(x_hbm.at[pl.ds(row_base, _FPG)], idx_v.at[0])
        fire_group(0)

        def body(g, carry):
            slot = lax.rem(g, 2)
            nslot = 1 - slot

            @pl.when(g + 1 < groups)
            def _():
                pltpu.async_copy(
                    x_hbm.at[pl.ds(row_base + (g + 1) * _FPG, _FPG)],
                    idx_v.at[nslot],
                    sem_i,
                )

            pltpu.make_async_copy(
                out_hbm.at[pl.ds(0, _GROUP), pl.ds(0, EMB)],
                rows_v.at[slot],
                sem_g,
            ).wait()

            @pl.when(g >= 1)
            def _():
                for src, dst in store_copies(nslot, 0):
                    pltpu.make_async_copy(src, dst, sem_o).wait()

            @pl.when(g + 1 < groups)
            def _():
                pltpu.make_async_copy(
                    x_hbm.at[pl.ds(row_base, _FPG)],
                    idx_v.at[nslot],
                    sem_i,
                ).wait()
                fire_group(nslot)

            for src, dst in store_copies(slot, g):
                pltpu.async_copy(src, dst, sem_o)
            return carry

        lax.fori_loop(0, groups, body, 0)

        for src, dst in store_copies((groups - 1) % 2, 0):
            pltpu.make_async_copy(src, dst, sem_o).wait()

    return gather(x2d, w)


def _tc_project(g4, m, b, l):
    bb = 16
    rows = bb * l
    nsteps = (b // bb) // _PACK

    def mm(g4_ref, m_ref, o_ref):
        for k in range(_PACK):
            acc = lax.dot_general(
                g4_ref[...],
                m_ref[k],
                (((1,), (0,)), ((), ())),
                preferred_element_type=jnp.float32,
            )
            o_ref[k] = acc.reshape(bb, l, HID)

    return pl.pallas_call(
        mm,
        grid=(nsteps,),
        in_specs=[
            pl.BlockSpec((rows, HID), lambda i: (i, 0)),
            pl.BlockSpec((_PACK, HID, HID), lambda i: (0, 0, 0)),
        ],
        out_specs=pl.BlockSpec((_PACK, bb, l, HID), lambda i: (0, i, 0, 0)),
        out_shape=jax.ShapeDtypeStruct((_PACK, b // _PACK, l, HID), jnp.float32),
    )(g4, m)


def kernel(x, W, We):
    b, l = x.shape
    n = b * l
    xf = x.reshape(-1).astype(jnp.int32)
    xp = (
        xf.reshape(_PACK, (n // _PACK) // _CHUNK, _CHUNK)
        .transpose(1, 0, 2)
        .reshape(n // _FIRE, _FIRE)
    )
    g4 = _sc_gather(xp, W)
    m = jnp.stack(
        [
            jnp.zeros((HID, HID), jnp.float32)
            .at[k * EMB:(k + 1) * EMB, :]
            .set(We.T)
            for k in range(_PACK)
        ]
    )
    out4 = _tc_project(g4, m, b, l)
    return out4.reshape(b, l, HID)
